# baseline (speedup 1.0000x reference)
import functools

import jax
import jax.numpy as jnp
from jax import lax
from jax.experimental import pallas as pl
from jax.experimental.pallas import tpu as pltpu
from jax.experimental.pallas import tpu_sc as plsc

HIDDEN = 2048
E = 8
DPG = 256
TOKENS = 4096
K = 2

TMR = 256
NBLK = TOKENS // TMR
TMG = 128
P = TOKENS * K + E * TMG
NT = P // TMG

NC, NS = 2, 16
NW = NC * NS
CPW = TOKENS // NW
CH = 32
NCH = CPW // CH
CH3 = 16
NCH3 = CPW // CH3


def _router_body(x_ref, wg_ref, eidx_ref, gate_ref, rw_ref, bh_ref):
    xb = x_ref[...]
    s = jax.nn.sigmoid(
        lax.dot_general(xb, wg_ref[...], (((1,), (1,)), ((), ())),
                        preferred_element_type=jnp.float32))
    iota = lax.broadcasted_iota(jnp.int32, (TMR, E), 1)
    g1 = jnp.max(s, axis=1, keepdims=True)
    i1 = jnp.min(jnp.where(s == g1, iota, E), axis=1, keepdims=True)
    s2 = jnp.where(iota == i1, -jnp.inf, s)
    g2 = jnp.max(s2, axis=1, keepdims=True)
    i2 = jnp.min(jnp.where(s2 == g2, iota, E), axis=1, keepdims=True)
    oh0 = (iota == i1).astype(jnp.int32)
    oh1 = (iota == i2).astype(jnp.int32)

    def _cumsum0(a):
        n = a.shape[0]
        sh = 1
        while sh < n:
            a = a + jnp.concatenate(
                [jnp.zeros((sh, a.shape[1]), a.dtype), a[:-sh]], axis=0)
            sh *= 2
        return a

    cum0 = _cumsum0(oh0)
    cum1 = _cumsum0(oh1)
    excl0 = cum0 - oh0 + cum1 - oh1
    excl1 = cum0 + cum1 - oh1
    rw0 = jnp.sum(oh0 * excl0, axis=1, keepdims=True)
    rw1 = jnp.sum(oh1 * excl1, axis=1, keepdims=True)
    eidx_ref[...] = jnp.concatenate([i1, i2], axis=1)
    gate_ref[...] = jnp.concatenate([g1, g2], axis=1)
    rw_ref[...] = jnp.concatenate([rw0, rw1], axis=1)
    bh_ref[...] = (cum0[-1:, :] + cum1[-1:, :]).reshape(1, 1, E)


@jax.jit
def _router(xf, Wg):
    return pl.pallas_call(
        _router_body,
        grid=(NBLK,),
        in_specs=[
            pl.BlockSpec((TMR, HIDDEN), lambda i: (i, 0)),
            pl.BlockSpec((E, HIDDEN), lambda i: (0, 0)),
        ],
        out_specs=[
            pl.BlockSpec((TMR, K), lambda i: (i, 0)),
            pl.BlockSpec((TMR, K), lambda i: (i, 0)),
            pl.BlockSpec((TMR, K), lambda i: (i, 0)),
            pl.BlockSpec((1, 1, E), lambda i: (i, 0, 0)),
        ],
        out_shape=[
            jax.ShapeDtypeStruct((TOKENS, K), jnp.int32),
            jax.ShapeDtypeStruct((TOKENS, K), jnp.float32),
            jax.ShapeDtypeStruct((TOKENS, K), jnp.int32),
            jax.ShapeDtypeStruct((NBLK, 1, E), jnp.int32),
        ],
        compiler_params=pltpu.CompilerParams(
            dimension_semantics=("arbitrary",),
        ),
    )(xf, Wg)


@functools.lru_cache(maxsize=None)
def _sc_kernels():
    mesh = plsc.VectorSubcoreMesh(core_axis_name="c", subcore_axis_name="s")

    @functools.partial(
        pl.kernel, mesh=mesh,
        out_type=[jax.ShapeDtypeStruct((P, HIDDEN), jnp.float32),
                  jax.ShapeDtypeStruct((P, 128), jnp.float32)],
        scratch_types=[pltpu.VMEM((CH,), jnp.int32),
                       pltpu.VMEM((CH, HIDDEN), jnp.float32),
                       pltpu.VMEM((CH, 128), jnp.float32)],
    )
    def dispatch(x_hbm, pos_hbm, gsp_hbm, xg_hbm, gs_hbm, idx_v, rows_v, g_v):
        w = lax.axis_index("s") * NC + lax.axis_index("c")
        base = w * CPW
        for c in range(NCH):
            pltpu.sync_copy(x_hbm.at[pl.ds(base + c * CH, CH)], rows_v)
            for r in range(K):
                pltpu.sync_copy(pos_hbm.at[w, r, c], idx_v)
                pltpu.sync_copy(gsp_hbm.at[w, r, c], g_v)
                pltpu.sync_copy(rows_v, xg_hbm.at[idx_v])
                pltpu.sync_copy(g_v, gs_hbm.at[idx_v])

    @functools.partial(
        pl.kernel, mesh=mesh,
        out_type=jax.ShapeDtypeStruct((P, DPG), jnp.float32),
        scratch_types=[pltpu.VMEM((CH,), jnp.int32),
                       pltpu.VMEM((CH,), jnp.int32),
                       pltpu.VMEM((CH, DPG), jnp.float32),
                       pltpu.VMEM((CH, DPG), jnp.float32)],
    )
    def combine_mid(dp_hbm, pos_hbm, dg_hbm, i0_v, i1_v, a_v, b_v):
        w = lax.axis_index("s") * NC + lax.axis_index("c")
        for c in range(NCH):
            pltpu.sync_copy(pos_hbm.at[w, 0, c], i0_v)
            pltpu.sync_copy(pos_hbm.at[w, 1, c], i1_v)
            pltpu.sync_copy(dp_hbm.at[i0_v], a_v)
            pltpu.sync_copy(dp_hbm.at[i1_v], b_v)

            def body(t, _):
                for seg in range(DPG // 16):
                    sl = pl.ds(seg * 16, 16)
                    a_v[t, sl] = a_v[t, sl] + b_v[t, sl]
                return 0

            lax.fori_loop(0, CH, body, 0)
            pltpu.sync_copy(a_v, dg_hbm.at[i0_v])
            pltpu.sync_copy(a_v, dg_hbm.at[i1_v])

    @functools.partial(
        pl.kernel, mesh=mesh,
        out_type=jax.ShapeDtypeStruct((TOKENS, HIDDEN), jnp.float32),
        scratch_types=[pltpu.VMEM((CH3,), jnp.int32),
                       pltpu.VMEM((CH3,), jnp.int32),
                       pltpu.VMEM((CH3, HIDDEN), jnp.float32),
                       pltpu.VMEM((CH3, HIDDEN), jnp.float32)],
    )
    def combine_out(up_hbm, pos_hbm, out_hbm, i0_v, i1_v, a_v, b_v):
        w = lax.axis_index("s") * NC + lax.axis_index("c")
        base = w * CPW
        for c in range(NCH3):
            pltpu.sync_copy(pos_hbm.at[w, 0, c], i0_v)
            pltpu.sync_copy(pos_hbm.at[w, 1, c], i1_v)
            pltpu.sync_copy(up_hbm.at[i0_v], a_v)
            pltpu.sync_copy(up_hbm.at[i1_v], b_v)

            def body(t, _):
                for seg in range(HIDDEN // 16):
                    sl = pl.ds(seg * 16, 16)
                    a_v[t, sl] = a_v[t, sl] + b_v[t, sl]
                return 0

            lax.fori_loop(0, CH3, body, 0)
            pltpu.sync_copy(a_v, out_hbm.at[pl.ds(base + c * CH3, CH3)])

    return dispatch, combine_mid, combine_out


def _grouped_body(te_ref, rows_ref, w_ref, gs_ref, o_ref):
    del te_ref
    t1 = lax.dot_general(rows_ref[...], w_ref[0], (((1,), (1,)), ((), ())),
                         preferred_element_type=jnp.float32)
    o_ref[...] = t1 * gs_ref[:, 0:1]


def _grouped_call(te, rows, w, gs, n_out):
    spec = pltpu.PrefetchScalarGridSpec(
        num_scalar_prefetch=1,
        grid=(NT,),
        in_specs=[
            pl.BlockSpec((TMG, rows.shape[1]), lambda i, te_r: (i, 0)),
            pl.BlockSpec((1,) + w.shape[1:], lambda i, te_r: (te_r[i], 0, 0)),
            pl.BlockSpec((TMG, 128), lambda i, te_r: (i, 0)),
        ],
        out_specs=pl.BlockSpec((TMG, n_out), lambda i, te_r: (i, 0)),
    )
    return pl.pallas_call(
        _grouped_body,
        grid_spec=spec,
        out_shape=jax.ShapeDtypeStruct((P, n_out), jnp.float32),
        compiler_params=pltpu.CompilerParams(
            dimension_semantics=("arbitrary",),
        ),
    )(te, rows, w, gs)


def kernel(x, Wg, Wd, Wu):
    b, l, d = x.shape
    xf = x.reshape(-1, d)

    eidx, gate, rw, bh3 = _router(xf, Wg)

    bh = bh3.reshape(NBLK, E)
    counts = bh.sum(axis=0)
    sizes = ((counts + TMG - 1) // TMG) * TMG
    offsets = jnp.concatenate(
        [jnp.zeros((1,), jnp.int32), jnp.cumsum(sizes)[:-1].astype(jnp.int32)])
    blockcarry = (jnp.cumsum(bh, axis=0) - bh).astype(jnp.int32)
    tok = jnp.arange(TOKENS, dtype=jnp.int32)[:, None] // TMR
    pos = offsets[eidx] + blockcarry[tok, eidx] + rw
    tile_start = jnp.arange(NT, dtype=jnp.int32) * TMG
    te = jnp.minimum(
        (tile_start[:, None] >= (offsets + sizes)[None, :]).sum(axis=1), E - 1
    ).astype(jnp.int32)

    posw = pos.reshape(NW, CPW, K).transpose(0, 2, 1)
    pos_c = posw.reshape(NW, K, NCH, CH)
    pos_c3 = posw.reshape(NW, K, NCH3, CH3)
    gsp = jnp.broadcast_to(
        gate.reshape(NW, CPW, K).transpose(0, 2, 1)[..., None],
        (NW, K, CPW, 128)).reshape(NW, K, NCH, CH, 128)

    dispatch, combine_mid, combine_out = _sc_kernels()
    xg, gs = dispatch(xf, pos_c, gsp)
    dp = _grouped_call(te, xg, Wd, gs, DPG)
    dg = combine_mid(dp, pos_c)
    up = _grouped_call(te, dg, Wu, gs, HIDDEN)
    out = combine_out(up, pos_c3)
    return out.reshape(b, l, d)


# speedup vs baseline: 4.4117x; 4.4117x over previous
import functools

import jax
import jax.numpy as jnp
from jax.experimental import pallas as pl
from jax.experimental.pallas import tpu as pltpu

HIDDEN = 2048
E = 8
DPG = 256
TOKENS = 4096
TM = 512


def _moe_block(x_ref, wg_ref, wd_ref, wu_ref, o_ref):
    xb = x_ref[...]
    s = jax.nn.sigmoid(
        jax.lax.dot_general(xb, wg_ref[...], (((1,), (1,)), ((), ())),
                            preferred_element_type=jnp.float32))
    iota = jax.lax.broadcasted_iota(jnp.int32, (TM, E), 1)
    g1 = jnp.max(s, axis=1, keepdims=True)
    i1 = jnp.min(jnp.where(s == g1, iota, E), axis=1, keepdims=True)
    s2 = jnp.where(iota == i1, -jnp.inf, s)
    g2 = jnp.max(s2, axis=1, keepdims=True)
    i2 = jnp.min(jnp.where(s2 == g2, iota, E), axis=1, keepdims=True)
    c = jnp.where(iota == i1, g1, 0.0) + jnp.where(iota == i2, g2, 0.0)

    t1 = jax.lax.dot_general(xb, wd_ref[...], (((1,), (1,)), ((), ())),
                             preferred_element_type=jnp.float32)
    down = c[:, 0:1] * t1[:, :DPG]
    for e in range(1, E):
        down = down + c[:, e:e + 1] * t1[:, e * DPG:(e + 1) * DPG]

    acc = jnp.zeros((TM, HIDDEN), jnp.float32)
    for e in range(E):
        ue = jax.lax.dot_general(down, wu_ref[e], (((1,), (1,)), ((), ())),
                                 preferred_element_type=jnp.float32)
        acc = acc + c[:, e:e + 1] * ue
    o_ref[...] = acc


@jax.jit
def _moe(xf, Wg, Wdf, Wu):
    nblk = TOKENS // TM
    return pl.pallas_call(
        _moe_block,
        grid=(nblk,),
        in_specs=[
            pl.BlockSpec((TM, HIDDEN), lambda i: (i, 0)),
            pl.BlockSpec((E, HIDDEN), lambda i: (0, 0)),
            pl.BlockSpec((E * DPG, HIDDEN), lambda i: (0, 0)),
            pl.BlockSpec((E, HIDDEN, DPG), lambda i: (0, 0, 0)),
        ],
        out_specs=pl.BlockSpec((TM, HIDDEN), lambda i: (i, 0)),
        out_shape=jax.ShapeDtypeStruct((TOKENS, HIDDEN), jnp.float32),
        compiler_params=pltpu.CompilerParams(
            dimension_semantics=("arbitrary",),
        ),
    )(xf, Wg, Wdf, Wu)


def kernel(x, Wg, Wd, Wu):
    b, l, d = x.shape
    xf = x.reshape(-1, d)
    Wdf = Wd.reshape(E * DPG, HIDDEN)
    out = _moe(xf, Wg, Wdf, Wu)
    return out.reshape(b, l, d)


# speedup vs baseline: 4.4151x; 1.0008x over previous
import jax
import jax.numpy as jnp
from jax.experimental import pallas as pl
from jax.experimental.pallas import tpu as pltpu

HIDDEN = 2048
E = 8
DPG = 256
TOKENS = 4096
TM = 512


def _moe_block(x_ref, wg_ref, wd_ref, wu_ref, o_ref):
    xb = x_ref[...]
    s = jax.nn.sigmoid(
        jax.lax.dot_general(xb, wg_ref[...], (((1,), (1,)), ((), ())),
                            preferred_element_type=jnp.float32))
    iota = jax.lax.broadcasted_iota(jnp.int32, (TM, E), 1)
    g1 = jnp.max(s, axis=1, keepdims=True)
    i1 = jnp.min(jnp.where(s == g1, iota, E), axis=1, keepdims=True)
    s2 = jnp.where(iota == i1, -jnp.inf, s)
    g2 = jnp.max(s2, axis=1, keepdims=True)
    i2 = jnp.min(jnp.where(s2 == g2, iota, E), axis=1, keepdims=True)
    c = jnp.where(iota == i1, g1, 0.0) + jnp.where(iota == i2, g2, 0.0)

    t1 = jax.lax.dot_general(xb, wd_ref[...], (((1,), (1,)), ((), ())),
                             preferred_element_type=jnp.float32)
    down = c[:, 0:1] * t1[:, :DPG]
    for e in range(1, E):
        down = down + c[:, e:e + 1] * t1[:, e * DPG:(e + 1) * DPG]

    acc = jnp.zeros((TM, HIDDEN), jnp.float32)
    for e in range(E):
        ue = jax.lax.dot_general(down, wu_ref[e], (((1,), (1,)), ((), ())),
                                 preferred_element_type=jnp.float32)
        acc = acc + c[:, e:e + 1] * ue
    o_ref[...] = acc


@jax.jit
def _moe(xf, Wg, Wdf, Wu):
    nblk = TOKENS // TM
    return pl.pallas_call(
        _moe_block,
        grid=(nblk,),
        in_specs=[
            pl.BlockSpec((TM, HIDDEN), lambda i: (i, 0)),
            pl.BlockSpec((E, HIDDEN), lambda i: (0, 0)),
            pl.BlockSpec((E * DPG, HIDDEN), lambda i: (0, 0)),
            pl.BlockSpec((E, HIDDEN, DPG), lambda i: (0, 0, 0)),
        ],
        out_specs=pl.BlockSpec((TM, HIDDEN), lambda i: (i, 0)),
        out_shape=jax.ShapeDtypeStruct((TOKENS, HIDDEN), jnp.float32),
        compiler_params=pltpu.CompilerParams(
            dimension_semantics=("arbitrary",),
        ),
    )(xf, Wg, Wdf, Wu)


def kernel(x, Wg, Wd, Wu):
    b, l, d = x.shape
    xf = x.reshape(-1, d)
    Wdf = Wd.reshape(E * DPG, HIDDEN)
    out = _moe(xf, Wg, Wdf, Wu)
    return out.reshape(b, l, d)
